# baseline (speedup 1.0000x reference)
import functools

import jax
import jax.numpy as jnp
from jax import lax
from jax.experimental import pallas as pl
from jax.experimental.pallas import tpu as pltpu
from jax.experimental.pallas import tpu_sc as plsc

K = 8192
D = 256
BETA = 0.25
CBLK = 1024
NJ = K // CBLK
NB = 8
T = 1024


def _argmin_body(z_ref, c_ref, idx_ref, zsq_s, minv_s, mini_s, csq_s, cbm_s):
    j = pl.program_id(0)
    b = pl.program_id(1)
    zb = z_ref[0]

    @pl.when(j == 0)
    def _():
        zsq_s[pl.ds(b, 1), :] = jnp.sum(zb * zb, axis=0, keepdims=True)
        minv_s[pl.ds(b, 1), :] = jnp.full((1, T), jnp.inf, jnp.float32)
        mini_s[pl.ds(b, 1), :] = jnp.zeros((1, T), jnp.int32)

    @pl.when(b == 0)
    def _():
        cb = c_ref[...]
        cbm_s[...] = -2.0 * cb
        csq_s[...] = jnp.sum(cb * cb, axis=1, keepdims=True)

    m = jnp.dot(cbm_s[...], zb, preferred_element_type=jnp.float32)
    zsqr = zsq_s[pl.ds(b, 1), :]

    minv8 = jnp.full((8, T), jnp.inf, jnp.float32)
    mini8 = jnp.zeros((8, T), jnp.int32)
    for c in range(CBLK // 8):
        mc = lax.slice(m, (c * 8, 0), (c * 8 + 8, T))
        csqc = csq_s[pl.ds(c * 8, 8), :]
        d = (zsqr + csqc) + mc
        better = d < minv8
        minv8 = jnp.where(better, d, minv8)
        mini8 = jnp.where(better, c, mini8)
    rows8 = mini8 * 8 + lax.broadcasted_iota(jnp.int32, (8, T), 0)
    bmin = jnp.min(minv8, axis=0, keepdims=True)
    bidx = jnp.min(jnp.where(minv8 == bmin, rows8, K), axis=0, keepdims=True) + j * CBLK

    better = bmin < minv_s[pl.ds(b, 1), :]
    mini_s[pl.ds(b, 1), :] = jnp.where(better, bidx, mini_s[pl.ds(b, 1), :])
    minv_s[pl.ds(b, 1), :] = jnp.where(better, bmin, minv_s[pl.ds(b, 1), :])

    @pl.when(j == NJ - 1)
    def _():
        idx_ref[0] = mini_s[pl.ds(b, 1), :]


def _argmin_indices(z3, codebook):
    out = pl.pallas_call(
        _argmin_body,
        grid=(NJ, NB),
        in_specs=[
            pl.BlockSpec((1, D, T), lambda j, b: (b, 0, 0)),
            pl.BlockSpec((CBLK, D), lambda j, b: (j, 0)),
        ],
        out_specs=pl.BlockSpec((1, 1, T), lambda j, b: (b, 0, 0)),
        out_shape=jax.ShapeDtypeStruct((NB, 1, T), jnp.int32),
        scratch_shapes=[
            pltpu.VMEM((NB, T), jnp.float32),
            pltpu.VMEM((NB, T), jnp.float32),
            pltpu.VMEM((NB, T), jnp.int32),
            pltpu.VMEM((CBLK, 1), jnp.float32),
            pltpu.VMEM((CBLK, D), jnp.float32),
        ],
    )(z3, codebook)
    return out.reshape(-1)


N_CORES = 2
N_SUB = 16
NW = N_CORES * N_SUB
TPW = K // NW
IPR = 128


def _sc_gather_hist_body(cb_hbm, idx_hbm, idx16_hbm, zeros_hbm, zq_hbm, hist_hbm,
                         idx_v, idx16_v, rows_v, hist_v, sem):
    c = lax.axis_index("c")
    s = lax.axis_index("s")
    w = c * N_SUB + s

    pltpu.sync_copy(zeros_hbm, hist_v)
    pltpu.sync_copy(idx_hbm.at[pl.ds(w * (TPW // IPR), TPW // IPR)], idx_v)
    pltpu.sync_copy(idx16_hbm.at[pl.ds(w * (TPW // 16), TPW // 16)], idx16_v)

    cp0 = pltpu.async_copy(cb_hbm.at[idx_v.at[0]], rows_v, sem)
    ones16 = jnp.ones((16,), jnp.int32)
    for i in range(TPW // 16):
        plsc.addupdate_scatter(hist_v, [idx16_v[i]], ones16)
    cp0.wait()
    pltpu.sync_copy(rows_v, zq_hbm.at[pl.ds(w * TPW, IPR)])
    pltpu.async_copy(cb_hbm.at[idx_v.at[1]], rows_v, sem).wait()
    pltpu.sync_copy(rows_v, zq_hbm.at[pl.ds(w * TPW + IPR, IPR)])

    pltpu.sync_copy(hist_v, hist_hbm.at[pl.ds(w * K, K)])


def _sc_gather_hist(codebook, vq_indices):
    idx2d = vq_indices.reshape(K // IPR, IPR)
    idx16 = vq_indices.reshape(K // 16, 16)
    mesh = plsc.VectorSubcoreMesh(core_axis_name="c", subcore_axis_name="s")
    fn = functools.partial(
        pl.kernel,
        out_type=[jax.ShapeDtypeStruct((K, D), jnp.float32),
                  jax.ShapeDtypeStruct((NW * K,), jnp.int32)],
        mesh=mesh,
        compiler_params=pltpu.CompilerParams(needs_layout_passes=False),
        scratch_types=[
            pltpu.VMEM((TPW // IPR, IPR), jnp.int32),
            pltpu.VMEM((TPW // 16, 16), jnp.int32),
            pltpu.VMEM((IPR, D), jnp.float32),
            pltpu.VMEM((K,), jnp.int32),
            pltpu.SemaphoreType.DMA,
        ],
    )(_sc_gather_hist_body)
    zeros_const = jnp.zeros((K,), jnp.int32)
    zq, hist = fn(codebook, idx2d, idx16, zeros_const)
    counts = jnp.sum(hist.reshape(NW, K), axis=0, dtype=jnp.int32)
    return zq, counts


VMAX = 128


def _sorted_desc(new_count, vq_count):
    def fast(new):
        vals = lax.broadcasted_iota(jnp.int32, (VMAX, 1), 0)
        hist = jnp.sum((new[None, :] == vals).astype(jnp.int32), axis=1)
        n_ge = new.shape[0] - jnp.concatenate([jnp.zeros((1,), jnp.int32),
                                               jnp.cumsum(hist)])[:VMAX]
        ks = lax.broadcasted_iota(jnp.int32, (1, new.shape[0]), 1)
        sorted_new = jnp.sum((n_ge[1:, None] > ks).astype(jnp.int32), axis=0)
        return sorted_new, sorted_new - 1

    def slow(new):
        sorted_new = jnp.sort(new)[::-1]
        cur = new - vq_count
        return sorted_new, jnp.sort(cur)[::-1]

    okay = (jnp.max(new_count) < VMAX) & jnp.all(vq_count == 1)
    return lax.cond(okay, fast, slow, new_count)


def _hist_stats(sorted_count, total):
    prob = sorted_count.astype(jnp.float32) / total
    c_sum = jnp.cumsum(prob)
    p10 = jnp.argmax(c_sum >= 0.1)
    p50 = jnp.argmax(c_sum >= 0.5)
    p90 = jnp.argmax(c_sum >= 0.9)
    return p10, p50, p90


def kernel(z, codebook, vq_count):
    z3 = z.reshape(NB, D, T)
    vq_indices = _argmin_indices(z3, codebook)

    z_quantized, vq_current_count = _sc_gather_hist(codebook, vq_indices)

    new_vq_count = vq_count + vq_current_count.astype(vq_count.dtype)
    sorted_new, sorted_cur = _sorted_desc(new_vq_count, vq_count)
    cur_p10, cur_p50, cur_p90 = _hist_stats(
        sorted_cur, jnp.sum(vq_current_count.astype(jnp.float32)))
    tot_p10, tot_p50, tot_p90 = _hist_stats(
        sorted_new, jnp.sum(new_vq_count.astype(jnp.float32)))
    top10 = sorted_new[:10]
    bot10 = sorted_new[K - 10:][::-1]

    zq_t = jnp.transpose(z_quantized.reshape(NB, 32, 32, D), (0, 3, 1, 2))
    q = z + (zq_t - z)
    codebook_loss = jnp.mean((zq_t - z) ** 2)
    commitment_loss = codebook_loss
    loss = codebook_loss + BETA * commitment_loss
    return (q, loss, codebook_loss, commitment_loss,
            cur_p10, cur_p50, cur_p90, tot_p10, tot_p50, tot_p90, top10, bot10)


# speedup vs baseline: 1.0314x; 1.0314x over previous
import functools

import jax
import jax.numpy as jnp
from jax import lax
from jax.experimental import pallas as pl
from jax.experimental.pallas import tpu as pltpu
from jax.experimental.pallas import tpu_sc as plsc

K = 8192
D = 256
BETA = 0.25
CBLK = 1024
NJ = K // CBLK
NB = 8
T = 1024


def _argmin_body(z_ref, c_ref, idx_ref, zsq_s, minv_s, mini_s, csq_s, cbm_s):
    j = pl.program_id(0)
    b = pl.program_id(1)
    zb = z_ref[b]

    @pl.when(j == 0)
    def _():
        zsq_s[pl.ds(b, 1), :] = jnp.sum(zb * zb, axis=0, keepdims=True)
        minv_s[pl.ds(b, 1), :] = jnp.full((1, T), jnp.inf, jnp.float32)
        mini_s[pl.ds(b, 1), :] = jnp.zeros((1, T), jnp.int32)

    @pl.when(b == 0)
    def _():
        cb = c_ref[...]
        cbm_s[...] = -2.0 * cb
        csq_s[...] = jnp.sum(cb * cb, axis=1, keepdims=True)

    m = jnp.dot(cbm_s[...], zb, preferred_element_type=jnp.float32)
    zsqr = zsq_s[pl.ds(b, 1), :]

    minv8 = jnp.full((8, T), jnp.inf, jnp.float32)
    mini8 = jnp.zeros((8, T), jnp.int32)
    for c in range(CBLK // 8):
        mc = lax.slice(m, (c * 8, 0), (c * 8 + 8, T))
        csqc = csq_s[pl.ds(c * 8, 8), :]
        d = (zsqr + csqc) + mc
        better = d < minv8
        minv8 = jnp.where(better, d, minv8)
        mini8 = jnp.where(better, c, mini8)
    rows8 = mini8 * 8 + lax.broadcasted_iota(jnp.int32, (8, T), 0)
    bmin = jnp.min(minv8, axis=0, keepdims=True)
    bidx = jnp.min(jnp.where(minv8 == bmin, rows8, K), axis=0, keepdims=True) + j * CBLK

    better = bmin < minv_s[pl.ds(b, 1), :]
    mini_s[pl.ds(b, 1), :] = jnp.where(better, bidx, mini_s[pl.ds(b, 1), :])
    minv_s[pl.ds(b, 1), :] = jnp.where(better, bmin, minv_s[pl.ds(b, 1), :])

    @pl.when(j == NJ - 1)
    def _():
        idx_ref[0] = mini_s[pl.ds(b, 1), :]


def _argmin_indices(z3, codebook):
    out = pl.pallas_call(
        _argmin_body,
        grid=(NJ, NB),
        in_specs=[
            pl.BlockSpec((NB, D, T), lambda j, b: (0, 0, 0)),
            pl.BlockSpec((CBLK, D), lambda j, b: (j, 0)),
        ],
        out_specs=pl.BlockSpec((1, 1, T), lambda j, b: (b, 0, 0)),
        out_shape=jax.ShapeDtypeStruct((NB, 1, T), jnp.int32),
        scratch_shapes=[
            pltpu.VMEM((NB, T), jnp.float32),
            pltpu.VMEM((NB, T), jnp.float32),
            pltpu.VMEM((NB, T), jnp.int32),
            pltpu.VMEM((CBLK, 1), jnp.float32),
            pltpu.VMEM((CBLK, D), jnp.float32),
        ],
    )(z3, codebook)
    return out.reshape(-1)


N_CORES = 2
N_SUB = 16
NW = N_CORES * N_SUB
TPW = K // NW
IPR = 128


def _sc_gather_hist_body(cb_hbm, idx_hbm, idx16_hbm, zeros_hbm, zq_hbm, hist_hbm,
                         idx_v, idx16_v, rows_v, hist_v, sem):
    c = lax.axis_index("c")
    s = lax.axis_index("s")
    w = c * N_SUB + s

    pltpu.sync_copy(zeros_hbm, hist_v)
    pltpu.sync_copy(idx_hbm.at[pl.ds(w * (TPW // IPR), TPW // IPR)], idx_v)
    pltpu.sync_copy(idx16_hbm.at[pl.ds(w * (TPW // 16), TPW // 16)], idx16_v)

    cp0 = pltpu.async_copy(cb_hbm.at[idx_v.at[0]], rows_v, sem)
    ones16 = jnp.ones((16,), jnp.int32)
    for i in range(TPW // 16):
        plsc.addupdate_scatter(hist_v, [idx16_v[i]], ones16)
    cp0.wait()
    pltpu.sync_copy(rows_v, zq_hbm.at[pl.ds(w * TPW, IPR)])
    pltpu.async_copy(cb_hbm.at[idx_v.at[1]], rows_v, sem).wait()
    pltpu.sync_copy(rows_v, zq_hbm.at[pl.ds(w * TPW + IPR, IPR)])

    pltpu.sync_copy(hist_v, hist_hbm.at[pl.ds(w * K, K)])


def _sc_gather_hist(codebook, vq_indices):
    idx2d = vq_indices.reshape(K // IPR, IPR)
    idx16 = vq_indices.reshape(K // 16, 16)
    mesh = plsc.VectorSubcoreMesh(core_axis_name="c", subcore_axis_name="s")
    fn = functools.partial(
        pl.kernel,
        out_type=[jax.ShapeDtypeStruct((K, D), jnp.float32),
                  jax.ShapeDtypeStruct((NW * K,), jnp.int32)],
        mesh=mesh,
        compiler_params=pltpu.CompilerParams(needs_layout_passes=False),
        scratch_types=[
            pltpu.VMEM((TPW // IPR, IPR), jnp.int32),
            pltpu.VMEM((TPW // 16, 16), jnp.int32),
            pltpu.VMEM((IPR, D), jnp.float32),
            pltpu.VMEM((K,), jnp.int32),
            pltpu.SemaphoreType.DMA,
        ],
    )(_sc_gather_hist_body)
    zeros_const = jnp.zeros((K,), jnp.int32)
    zq, hist = fn(codebook, idx2d, idx16, zeros_const)
    counts = jnp.sum(hist.reshape(NW, K), axis=0, dtype=jnp.int32)
    return zq, counts


VMAX = 128


def _sorted_desc(new_count, vq_count):
    def fast(new):
        vals = lax.broadcasted_iota(jnp.int32, (VMAX, 1), 0)
        hist = jnp.sum((new[None, :] == vals).astype(jnp.int32), axis=1)
        n_ge = new.shape[0] - jnp.concatenate([jnp.zeros((1,), jnp.int32),
                                               jnp.cumsum(hist)])[:VMAX]
        ks = lax.broadcasted_iota(jnp.int32, (1, new.shape[0]), 1)
        sorted_new = jnp.sum((n_ge[1:, None] > ks).astype(jnp.int32), axis=0)
        return sorted_new, sorted_new - 1

    def slow(new):
        sorted_new = jnp.sort(new)[::-1]
        cur = new - vq_count
        return sorted_new, jnp.sort(cur)[::-1]

    okay = (jnp.max(new_count) < VMAX) & jnp.all(vq_count == 1)
    return lax.cond(okay, fast, slow, new_count)


def _hist_stats(sorted_count, total):
    prob = sorted_count.astype(jnp.float32) / total
    c_sum = jnp.cumsum(prob)
    p10 = jnp.argmax(c_sum >= 0.1)
    p50 = jnp.argmax(c_sum >= 0.5)
    p90 = jnp.argmax(c_sum >= 0.9)
    return p10, p50, p90


def kernel(z, codebook, vq_count):
    z3 = z.reshape(NB, D, T)
    vq_indices = _argmin_indices(z3, codebook)

    z_quantized, vq_current_count = _sc_gather_hist(codebook, vq_indices)

    new_vq_count = vq_count + vq_current_count.astype(vq_count.dtype)
    sorted_new, sorted_cur = _sorted_desc(new_vq_count, vq_count)
    cur_p10, cur_p50, cur_p90 = _hist_stats(
        sorted_cur, jnp.sum(vq_current_count.astype(jnp.float32)))
    tot_p10, tot_p50, tot_p90 = _hist_stats(
        sorted_new, jnp.sum(new_vq_count.astype(jnp.float32)))
    top10 = sorted_new[:10]
    bot10 = sorted_new[K - 10:][::-1]

    zq_t = jnp.transpose(z_quantized.reshape(NB, 32, 32, D), (0, 3, 1, 2))
    q = z + (zq_t - z)
    codebook_loss = jnp.mean((zq_t - z) ** 2)
    commitment_loss = codebook_loss
    loss = codebook_loss + BETA * commitment_loss
    return (q, loss, codebook_loss, commitment_loss,
            cur_p10, cur_p50, cur_p90, tot_p10, tot_p50, tot_p90, top10, bot10)


# speedup vs baseline: 1.1611x; 1.1257x over previous
import functools

import jax
import jax.numpy as jnp
from jax import lax
from jax.experimental import pallas as pl
from jax.experimental.pallas import tpu as pltpu
from jax.experimental.pallas import tpu_sc as plsc

K = 8192
D = 256
BETA = 0.25
CBLK = 4096
NJ = K // CBLK
NB = 8
T = 1024


def _argmin_body(z_ref, c_ref, idx_ref, zsq_s, minv_s, mini_s, csq_s, cbm_s):
    j = pl.program_id(0)
    b = pl.program_id(1)
    zb = z_ref[b]

    @pl.when(j == 0)
    def _():
        zsq_s[pl.ds(b, 1), :] = jnp.sum(zb * zb, axis=0, keepdims=True)
        minv_s[pl.ds(b, 1), :] = jnp.full((1, T), jnp.inf, jnp.float32)
        mini_s[pl.ds(b, 1), :] = jnp.zeros((1, T), jnp.int32)

    @pl.when(b == 0)
    def _():
        cb = c_ref[...]
        cbm_s[...] = -2.0 * cb
        csq_s[...] = jnp.sum(cb * cb, axis=1, keepdims=True)

    m = jnp.dot(cbm_s[...], zb, preferred_element_type=jnp.float32)
    zsqr = zsq_s[pl.ds(b, 1), :]

    minv8 = jnp.full((8, T), jnp.inf, jnp.float32)
    mini8 = jnp.zeros((8, T), jnp.int32)
    for c in range(CBLK // 8):
        mc = lax.slice(m, (c * 8, 0), (c * 8 + 8, T))
        csqc = csq_s[pl.ds(c * 8, 8), :]
        d = (zsqr + csqc) + mc
        better = d < minv8
        minv8 = jnp.where(better, d, minv8)
        mini8 = jnp.where(better, c, mini8)
    rows8 = mini8 * 8 + lax.broadcasted_iota(jnp.int32, (8, T), 0)
    bmin = jnp.min(minv8, axis=0, keepdims=True)
    bidx = jnp.min(jnp.where(minv8 == bmin, rows8, K), axis=0, keepdims=True) + j * CBLK

    better = bmin < minv_s[pl.ds(b, 1), :]
    mini_s[pl.ds(b, 1), :] = jnp.where(better, bidx, mini_s[pl.ds(b, 1), :])
    minv_s[pl.ds(b, 1), :] = jnp.where(better, bmin, minv_s[pl.ds(b, 1), :])

    @pl.when(j == NJ - 1)
    def _():
        idx_ref[0] = mini_s[pl.ds(b, 1), :]


def _argmin_indices(z3, codebook):
    out = pl.pallas_call(
        _argmin_body,
        grid=(NJ, NB),
        in_specs=[
            pl.BlockSpec((NB, D, T), lambda j, b: (0, 0, 0)),
            pl.BlockSpec((CBLK, D), lambda j, b: (j, 0)),
        ],
        out_specs=pl.BlockSpec((1, 1, T), lambda j, b: (b, 0, 0)),
        out_shape=jax.ShapeDtypeStruct((NB, 1, T), jnp.int32),
        scratch_shapes=[
            pltpu.VMEM((NB, T), jnp.float32),
            pltpu.VMEM((NB, T), jnp.float32),
            pltpu.VMEM((NB, T), jnp.int32),
            pltpu.VMEM((CBLK, 1), jnp.float32),
            pltpu.VMEM((CBLK, D), jnp.float32),
        ],
    )(z3, codebook)
    return out.reshape(-1)


N_CORES = 2
N_SUB = 16
NW = N_CORES * N_SUB
TPW = K // NW
IPR = 128


def _sc_gather_hist_body(cb_hbm, idx_hbm, idx16_hbm, zeros_hbm, zq_hbm, hist_hbm,
                         idx_v, idx16_v, rows_v, hist_v, sem):
    c = lax.axis_index("c")
    s = lax.axis_index("s")
    w = c * N_SUB + s

    pltpu.sync_copy(zeros_hbm, hist_v)
    pltpu.sync_copy(idx_hbm.at[pl.ds(w * (TPW // IPR), TPW // IPR)], idx_v)
    pltpu.sync_copy(idx16_hbm.at[pl.ds(w * (TPW // 16), TPW // 16)], idx16_v)

    cp0 = pltpu.async_copy(cb_hbm.at[idx_v.at[0]], rows_v, sem)
    ones16 = jnp.ones((16,), jnp.int32)
    for i in range(TPW // 16):
        plsc.addupdate_scatter(hist_v, [idx16_v[i]], ones16)
    cp0.wait()
    pltpu.sync_copy(rows_v, zq_hbm.at[pl.ds(w * TPW, IPR)])
    pltpu.async_copy(cb_hbm.at[idx_v.at[1]], rows_v, sem).wait()
    pltpu.sync_copy(rows_v, zq_hbm.at[pl.ds(w * TPW + IPR, IPR)])

    pltpu.sync_copy(hist_v, hist_hbm.at[pl.ds(w * K, K)])


def _sc_gather_hist(codebook, vq_indices):
    idx2d = vq_indices.reshape(K // IPR, IPR)
    idx16 = vq_indices.reshape(K // 16, 16)
    mesh = plsc.VectorSubcoreMesh(core_axis_name="c", subcore_axis_name="s")
    fn = functools.partial(
        pl.kernel,
        out_type=[jax.ShapeDtypeStruct((K, D), jnp.float32),
                  jax.ShapeDtypeStruct((NW * K,), jnp.int32)],
        mesh=mesh,
        compiler_params=pltpu.CompilerParams(needs_layout_passes=False),
        scratch_types=[
            pltpu.VMEM((TPW // IPR, IPR), jnp.int32),
            pltpu.VMEM((TPW // 16, 16), jnp.int32),
            pltpu.VMEM((IPR, D), jnp.float32),
            pltpu.VMEM((K,), jnp.int32),
            pltpu.SemaphoreType.DMA,
        ],
    )(_sc_gather_hist_body)
    zeros_const = jnp.zeros((K,), jnp.int32)
    zq, hist = fn(codebook, idx2d, idx16, zeros_const)
    counts = jnp.sum(hist.reshape(NW, K), axis=0, dtype=jnp.int32)
    return zq, counts


VMAX = 128


def _sorted_desc(new_count, vq_count):
    def fast(new):
        vals = lax.broadcasted_iota(jnp.int32, (VMAX, 1), 0)
        hist = jnp.sum((new[None, :] == vals).astype(jnp.int32), axis=1)
        n_ge = new.shape[0] - jnp.concatenate([jnp.zeros((1,), jnp.int32),
                                               jnp.cumsum(hist)])[:VMAX]
        ks = lax.broadcasted_iota(jnp.int32, (1, new.shape[0]), 1)
        sorted_new = jnp.sum((n_ge[1:, None] > ks).astype(jnp.int32), axis=0)
        return sorted_new, sorted_new - 1

    def slow(new):
        sorted_new = jnp.sort(new)[::-1]
        cur = new - vq_count
        return sorted_new, jnp.sort(cur)[::-1]

    okay = (jnp.max(new_count) < VMAX) & jnp.all(vq_count == 1)
    return lax.cond(okay, fast, slow, new_count)


def _hist_stats(sorted_count, total):
    prob = sorted_count.astype(jnp.float32) / total
    c_sum = jnp.cumsum(prob)
    p10 = jnp.argmax(c_sum >= 0.1)
    p50 = jnp.argmax(c_sum >= 0.5)
    p90 = jnp.argmax(c_sum >= 0.9)
    return p10, p50, p90


def kernel(z, codebook, vq_count):
    z3 = z.reshape(NB, D, T)
    vq_indices = _argmin_indices(z3, codebook)

    z_quantized, vq_current_count = _sc_gather_hist(codebook, vq_indices)

    new_vq_count = vq_count + vq_current_count.astype(vq_count.dtype)
    sorted_new, sorted_cur = _sorted_desc(new_vq_count, vq_count)
    cur_p10, cur_p50, cur_p90 = _hist_stats(
        sorted_cur, jnp.sum(vq_current_count.astype(jnp.float32)))
    tot_p10, tot_p50, tot_p90 = _hist_stats(
        sorted_new, jnp.sum(new_vq_count.astype(jnp.float32)))
    top10 = sorted_new[:10]
    bot10 = sorted_new[K - 10:][::-1]

    zq_t = jnp.transpose(z_quantized.reshape(NB, 32, 32, D), (0, 3, 1, 2))
    q = z + (zq_t - z)
    codebook_loss = jnp.mean((zq_t - z) ** 2)
    commitment_loss = codebook_loss
    loss = codebook_loss + BETA * commitment_loss
    return (q, loss, codebook_loss, commitment_loss,
            cur_p10, cur_p50, cur_p90, tot_p10, tot_p50, tot_p90, top10, bot10)


# speedup vs baseline: 1.1630x; 1.0017x over previous
import functools

import jax
import jax.numpy as jnp
from jax import lax
from jax.experimental import pallas as pl
from jax.experimental.pallas import tpu as pltpu
from jax.experimental.pallas import tpu_sc as plsc

K = 8192
D = 256
BETA = 0.25
CBLK = 4096
NJ = K // CBLK
NB = 8
T = 1024


def _argmin_body(z_ref, c_ref, idx_ref, zsq_s, minv_s, mini_s, csq_s, cbm_s):
    j = pl.program_id(0)
    b = pl.program_id(1)
    zb = z_ref[b]

    @pl.when(j == 0)
    def _():
        zsq_s[pl.ds(b, 1), :] = jnp.sum(zb * zb, axis=0, keepdims=True)
        minv_s[pl.ds(b, 1), :] = jnp.full((1, T), jnp.inf, jnp.float32)
        mini_s[pl.ds(b, 1), :] = jnp.zeros((1, T), jnp.int32)

    @pl.when(b == 0)
    def _():
        cb = c_ref[...]
        cbm_s[...] = -2.0 * cb
        csq_s[...] = jnp.sum(cb * cb, axis=1, keepdims=True)

    m = jnp.dot(cbm_s[...], zb, preferred_element_type=jnp.float32)
    zsqr = zsq_s[pl.ds(b, 1), :]

    minv8 = jnp.full((8, T), jnp.inf, jnp.float32)
    mini8 = jnp.zeros((8, T), jnp.int32)
    for c in range(CBLK // 8):
        mc = lax.slice(m, (c * 8, 0), (c * 8 + 8, T))
        csqc = csq_s[pl.ds(c * 8, 8), :]
        d = (zsqr + csqc) + mc
        better = d < minv8
        minv8 = jnp.where(better, d, minv8)
        mini8 = jnp.where(better, c, mini8)
    rows8 = mini8 * 8 + lax.broadcasted_iota(jnp.int32, (8, T), 0)
    bmin = jnp.min(minv8, axis=0, keepdims=True)
    bidx = jnp.min(jnp.where(minv8 == bmin, rows8, K), axis=0, keepdims=True) + j * CBLK

    better = bmin < minv_s[pl.ds(b, 1), :]
    mini_s[pl.ds(b, 1), :] = jnp.where(better, bidx, mini_s[pl.ds(b, 1), :])
    minv_s[pl.ds(b, 1), :] = jnp.where(better, bmin, minv_s[pl.ds(b, 1), :])

    @pl.when(j == NJ - 1)
    def _():
        idx_ref[0] = mini_s[pl.ds(b, 1), :]


def _argmin_indices(z3, codebook):
    out = pl.pallas_call(
        _argmin_body,
        grid=(NJ, NB),
        in_specs=[
            pl.BlockSpec((NB, D, T), lambda j, b: (0, 0, 0)),
            pl.BlockSpec((CBLK, D), lambda j, b: (j, 0)),
        ],
        out_specs=pl.BlockSpec((1, 1, T), lambda j, b: (b, 0, 0)),
        out_shape=jax.ShapeDtypeStruct((NB, 1, T), jnp.int32),
        scratch_shapes=[
            pltpu.VMEM((NB, T), jnp.float32),
            pltpu.VMEM((NB, T), jnp.float32),
            pltpu.VMEM((NB, T), jnp.int32),
            pltpu.VMEM((CBLK, 1), jnp.float32),
            pltpu.VMEM((CBLK, D), jnp.float32),
        ],
    )(z3, codebook)
    return out.reshape(-1)


N_CORES = 2
N_SUB = 16
NW = N_CORES * N_SUB
TPW = K // NW
IPR = 128


def _sc_gather_hist_body(cb_hbm, idx_hbm, idx16_hbm, zq_hbm, hist_hbm,
                         idx_v, idx16_v, rows_v, hist_v, sem):
    c = lax.axis_index("c")
    s = lax.axis_index("s")
    w = c * N_SUB + s

    pltpu.sync_copy(idx_hbm.at[pl.ds(w * (TPW // IPR), TPW // IPR)], idx_v)
    zrow = jnp.zeros((16,), jnp.int32)

    def zstep(i, _):
        hist_v[pl.ds(i * 16, 16)] = zrow
        return 0
    lax.fori_loop(0, K // 16, zstep, 0)
    pltpu.sync_copy(idx16_hbm.at[pl.ds(w * (TPW // 16), TPW // 16)], idx16_v)

    cp0 = pltpu.async_copy(cb_hbm.at[idx_v.at[0]], rows_v, sem)
    ones16 = jnp.ones((16,), jnp.int32)
    for i in range(TPW // 16):
        plsc.addupdate_scatter(hist_v, [idx16_v[i]], ones16)
    cp0.wait()
    pltpu.sync_copy(rows_v, zq_hbm.at[pl.ds(w * TPW, IPR)])
    pltpu.async_copy(cb_hbm.at[idx_v.at[1]], rows_v, sem).wait()
    pltpu.sync_copy(rows_v, zq_hbm.at[pl.ds(w * TPW + IPR, IPR)])

    pltpu.sync_copy(hist_v, hist_hbm.at[pl.ds(w * K, K)])


def _sc_gather_hist(codebook, vq_indices):
    idx2d = vq_indices.reshape(K // IPR, IPR)
    idx16 = vq_indices.reshape(K // 16, 16)
    mesh = plsc.VectorSubcoreMesh(core_axis_name="c", subcore_axis_name="s")
    fn = functools.partial(
        pl.kernel,
        out_type=[jax.ShapeDtypeStruct((K, D), jnp.float32),
                  jax.ShapeDtypeStruct((NW * K,), jnp.int32)],
        mesh=mesh,
        compiler_params=pltpu.CompilerParams(needs_layout_passes=False),
        scratch_types=[
            pltpu.VMEM((TPW // IPR, IPR), jnp.int32),
            pltpu.VMEM((TPW // 16, 16), jnp.int32),
            pltpu.VMEM((IPR, D), jnp.float32),
            pltpu.VMEM((K,), jnp.int32),
            pltpu.SemaphoreType.DMA,
        ],
    )(_sc_gather_hist_body)
    zq, hist = fn(codebook, idx2d, idx16)
    counts = jnp.sum(hist.reshape(NW, K), axis=0, dtype=jnp.int32)
    return zq, counts


VMAX = 128


def _sorted_desc(new_count, vq_count):
    def fast(new):
        vals = lax.broadcasted_iota(jnp.int32, (VMAX, 1), 0)
        hist = jnp.sum((new[None, :] == vals).astype(jnp.int32), axis=1)
        n_ge = new.shape[0] - jnp.concatenate([jnp.zeros((1,), jnp.int32),
                                               jnp.cumsum(hist)])[:VMAX]
        ks = lax.broadcasted_iota(jnp.int32, (1, new.shape[0]), 1)
        sorted_new = jnp.sum((n_ge[1:, None] > ks).astype(jnp.int32), axis=0)
        return sorted_new, sorted_new - 1

    def slow(new):
        sorted_new = jnp.sort(new)[::-1]
        cur = new - vq_count
        return sorted_new, jnp.sort(cur)[::-1]

    okay = (jnp.max(new_count) < VMAX) & jnp.all(vq_count == 1)
    return lax.cond(okay, fast, slow, new_count)


def _hist_stats(sorted_count, total):
    prob = sorted_count.astype(jnp.float32) / total
    c_sum = jnp.cumsum(prob)
    p10 = jnp.argmax(c_sum >= 0.1)
    p50 = jnp.argmax(c_sum >= 0.5)
    p90 = jnp.argmax(c_sum >= 0.9)
    return p10, p50, p90


def kernel(z, codebook, vq_count):
    z3 = z.reshape(NB, D, T)
    vq_indices = _argmin_indices(z3, codebook)

    z_quantized, vq_current_count = _sc_gather_hist(codebook, vq_indices)

    new_vq_count = vq_count + vq_current_count.astype(vq_count.dtype)
    sorted_new, sorted_cur = _sorted_desc(new_vq_count, vq_count)
    cur_p10, cur_p50, cur_p90 = _hist_stats(
        sorted_cur, jnp.sum(vq_current_count.astype(jnp.float32)))
    tot_p10, tot_p50, tot_p90 = _hist_stats(
        sorted_new, jnp.sum(new_vq_count.astype(jnp.float32)))
    top10 = sorted_new[:10]
    bot10 = sorted_new[K - 10:][::-1]

    zq_t = jnp.transpose(z_quantized.reshape(NB, 32, 32, D), (0, 3, 1, 2))
    q = z + (zq_t - z)
    codebook_loss = jnp.mean((zq_t - z) ** 2)
    commitment_loss = codebook_loss
    loss = codebook_loss + BETA * commitment_loss
    return (q, loss, codebook_loss, commitment_loss,
            cur_p10, cur_p50, cur_p90, tot_p10, tot_p50, tot_p90, top10, bot10)
